# baseline (speedup 1.0000x reference)
import functools

import jax
import jax.numpy as jnp
from jax import lax
from jax.experimental import pallas as pl
from jax.experimental.pallas import tpu as pltpu
from jax.experimental.pallas import tpu_sc as plsc

HDIM = 128
EPS_BN = 1e-5
EPS_SQRT = 1e-6
BN_SCALE = 1.0 / (1.0 + EPS_BN) ** 0.5

_PREC = jax.lax.Precision.DEFAULT


def _pair_mlps_body(x_ref, w1a_ref, b1a_ref, g1a_ref, be1a_ref, w1b_ref, b1b_ref,
                    w2a_ref, b2a_ref, g2a_ref, be2a_ref, w2b_ref, b2b_ref,
                    h1_ref, h2_ref):
    x = x_ref[...]
    t1 = jnp.dot(x, w1a_ref[...], precision=_PREC) + b1a_ref[...]
    t1 = t1 * (BN_SCALE * g1a_ref[...]) + be1a_ref[...]
    t1 = jnp.maximum(t1, 0.0)
    h1_ref[...] = jnp.dot(t1, w1b_ref[...], precision=_PREC) + b1b_ref[...]
    t2 = jnp.dot(x, w2a_ref[...], precision=_PREC) + b2a_ref[...]
    t2 = t2 * (BN_SCALE * g2a_ref[...]) + be2a_ref[...]
    t2 = jnp.maximum(t2, 0.0)
    h2_ref[...] = jnp.dot(t2, w2b_ref[...], precision=_PREC) + b2b_ref[...]


def _update_body(x_ref, agg_ref, wu1x_ref, wu1a_ref, bu1_ref, gu_ref, beu_ref,
                 wu2_ref, bu2_ref, out_ref):
    x = x_ref[...]
    a = agg_ref[...]
    a = jnp.sqrt(jnp.maximum(a, 0.0) + EPS_SQRT) - jnp.sqrt(jnp.maximum(-a, 0.0) + EPS_SQRT)
    t = (jnp.dot(x, wu1x_ref[...], precision=_PREC)
         + jnp.dot(a, wu1a_ref[...], precision=_PREC) + bu1_ref[...])
    t = t * (BN_SCALE * gu_ref[...]) + beu_ref[...]
    t = jnp.maximum(t, 0.0)
    out_ref[...] = jnp.dot(t, wu2_ref[...], precision=_PREC) + bu2_ref[...] + x


def _full(shape):
    return pl.BlockSpec(shape, lambda i: (0,) * len(shape))


def _pair_mlps(x, w1a, b1a, g1a, be1a, w1b, b1b, w2a, b2a, g2a, be2a, w2b, b2b):
    P = x.shape[0]
    BP = 2000
    grid = (P // BP,)
    row = pl.BlockSpec((BP, HDIM), lambda i: (i, 0))
    mat = _full((HDIM, HDIM))
    vec = _full((HDIM,))
    return pl.pallas_call(
        _pair_mlps_body,
        grid=grid,
        in_specs=[row, mat, vec, vec, vec, mat, vec, mat, vec, vec, vec, mat, vec],
        out_specs=[row, row],
        out_shape=[jax.ShapeDtypeStruct((P, HDIM), jnp.float32)] * 2,
    )(x, w1a, b1a, g1a, be1a, w1b, b1b, w2a, b2a, g2a, be2a, w2b, b2b)


def _update(x, agg, wu1, bu1, gu, beu, wu2, bu2):
    P = x.shape[0]
    BP = 2000
    grid = (P // BP,)
    row = pl.BlockSpec((BP, HDIM), lambda i: (i, 0))
    mat = _full((HDIM, HDIM))
    vec = _full((HDIM,))
    return pl.pallas_call(
        _update_body,
        grid=grid,
        in_specs=[row, row, mat, mat, vec, vec, vec, mat, vec],
        out_specs=row,
        out_shape=jax.ShapeDtypeStruct((P, HDIM), jnp.float32),
    )(x, agg, wu1[:HDIM], wu1[HDIM:], bu1, gu, beu, wu2, bu2)


_T_TOTAL = 1280000
_T_TILE = _T_TOTAL // 16
_SLAB = 2000
_NVREG = _SLAB // 16
_NTRIAD = 13
_NGRP = 20
_PG = 160000 // _NGRP
_PG_PAD = 8192
_TRASH = _PG
_CAP = 2176
_BATCH = 64
_BSH = 6


def _sc_body(h1, h2, i0, i1, i2, agg, spmem,
             sa0, sa1, sa2, sb0, sb1, sb2, sc0_, sc1_, sc2_,
             cc0, cc1, cc2,
             b0A, b1A, b2A, b0B, b1B, b2B,
             g1A, g2A, g1B, g2B,
             ma0, ma1, ma2, mb0, mb1, mb2, mc0, mc1, mc2,
             mg1A, mg2A, mg1B, mg2B):
    c = lax.axis_index("c")
    s = lax.axis_index("s")
    tile_base = s * _T_TILE

    def issue_slab_a(sl):
        base = tile_base + sl * _SLAB
        pltpu.async_copy(i0.at[pl.ds(base, _SLAB)], sa0, ma0)
        pltpu.async_copy(i1.at[pl.ds(base, _SLAB)], sa1, ma1)
        pltpu.async_copy(i2.at[pl.ds(base, _SLAB)], sa2, ma2)

    def wait_slab_a():
        pltpu.make_async_copy(i0.at[pl.ds(0, _SLAB)], sa0, ma0).wait()
        pltpu.make_async_copy(i1.at[pl.ds(0, _SLAB)], sa1, ma1).wait()
        pltpu.make_async_copy(i2.at[pl.ds(0, _SLAB)], sa2, ma2).wait()

    def issue_slab_b(sl):
        base = tile_base + sl * _SLAB
        pltpu.async_copy(i0.at[pl.ds(base, _SLAB)], sb0, mb0)
        pltpu.async_copy(i1.at[pl.ds(base, _SLAB)], sb1, mb1)
        pltpu.async_copy(i2.at[pl.ds(base, _SLAB)], sb2, mb2)

    def wait_slab_b():
        pltpu.make_async_copy(i0.at[pl.ds(0, _SLAB)], sb0, mb0).wait()
        pltpu.make_async_copy(i1.at[pl.ds(0, _SLAB)], sb1, mb1).wait()
        pltpu.make_async_copy(i2.at[pl.ds(0, _SLAB)], sb2, mb2).wait()

    def issue_slab_c(sl):
        base = tile_base + sl * _SLAB
        pltpu.async_copy(i0.at[pl.ds(base, _SLAB)], sc0_, mc0)
        pltpu.async_copy(i1.at[pl.ds(base, _SLAB)], sc1_, mc1)
        pltpu.async_copy(i2.at[pl.ds(base, _SLAB)], sc2_, mc2)

    def wait_slab_c():
        pltpu.make_async_copy(i0.at[pl.ds(0, _SLAB)], sc0_, mc0).wait()
        pltpu.make_async_copy(i1.at[pl.ds(0, _SLAB)], sc1_, mc1).wait()
        pltpu.make_async_copy(i2.at[pl.ds(0, _SLAB)], sc2_, mc2).wait()

    def complete_a(bq):
        pltpu.make_async_copy(h1.at[b1A], g1A, mg1A).wait()
        pltpu.make_async_copy(h2.at[b2A], g2A, mg2A).wait()

        def mul_row(r, _):
            for q in range(8):
                d = pl.ds(q * 16, 16)
                g1A[r, d] = g1A[r, d] * g2A[r, d]
            return 0
        lax.fori_loop(0, _BATCH, mul_row, 0)
        pltpu.sync_copy(g1A, spmem.at[b0A], add=True)

    def complete_b(bq):
        pltpu.make_async_copy(h1.at[b1B], g1B, mg1B).wait()
        pltpu.make_async_copy(h2.at[b2B], g2B, mg2B).wait()

        def mul_row(r, _):
            for q in range(8):
                d = pl.ds(q * 16, 16)
                g1B[r, d] = g1B[r, d] * g2B[r, d]
            return 0
        lax.fori_loop(0, _BATCH, mul_row, 0)
        pltpu.sync_copy(g1B, spmem.at[b0B], add=True)

    def issue_a(start):
        for k in range(_BATCH // 16):
            d = pl.ds(k * 16, 16)
            t = pl.ds(start + k * 16, 16)
            b0A[d] = cc0[t]
            b1A[d] = cc1[t]
            b2A[d] = cc2[t]
        pltpu.async_copy(h1.at[b1A], g1A, mg1A)
        pltpu.async_copy(h2.at[b2A], g2A, mg2A)

    def issue_b(start):
        for k in range(_BATCH // 16):
            d = pl.ds(k * 16, 16)
            t = pl.ds(start + k * 16, 16)
            b0B[d] = cc0[t]
            b1B[d] = cc1[t]
            b2B[d] = cc2[t]
        pltpu.async_copy(h1.at[b1B], g1B, mg1B)
        pltpu.async_copy(h2.at[b2B], g2B, mg2B)

    def batch_step(start, bq):
        @pl.when((bq & 1) == 0)
        def _():
            @pl.when(bq >= 2)
            def _():
                complete_a(bq)
            issue_a(start)

        @pl.when((bq & 1) == 1)
        def _():
            @pl.when(bq >= 2)
            def _():
                complete_b(bq)
            issue_b(start)
        return bq + 1

    def pass_body(g, _):
        lo = (c * (_NGRP // 2) + g) * _PG

        def zero_row(r, _):
            for q in range(8):
                g1A[r, pl.ds(q * 16, 16)] = jnp.zeros((16,), jnp.float32)
            return 0
        lax.fori_loop(0, _BATCH, zero_row, 0)
        for k in range(512 // _BATCH):
            pltpu.sync_copy(g1A, spmem.at[pl.ds(s * 512 + k * _BATCH, _BATCH)])
        plsc.subcore_barrier()

        def compact_and_batch(s0, s1, s2, curv, bq):
            def vreg_body(i, curv):
                off = pl.ds(i * 16, 16)
                d0 = s0[off] - lo
                m = d0.astype(jnp.uint32) < jnp.uint32(_PG)
                mi = m.astype(jnp.int32)
                pos = curv + plsc.cumsum(mi) - mi
                plsc.store_scatter(cc0, [pos], d0, mask=m)
                plsc.store_scatter(cc1, [pos], s1[off], mask=m)
                plsc.store_scatter(cc2, [pos], s2[off], mask=m)
                return curv + plsc.all_reduce_population_count(m)
            curv = lax.fori_loop(0, _NVREG, vreg_body, curv)
            cur = curv[0]

            nfull = cur >> _BSH

            def bb(j, bq):
                return batch_step(j * _BATCH, bq)
            bq = lax.fori_loop(0, nfull, bb, bq)

            tail = nfull << _BSH

            @pl.when(nfull > 0)
            def _():
                for k in range(_BATCH // 16):
                    d = pl.ds(k * 16, 16)
                    t = pl.ds(tail + k * 16, 16)
                    cc0[d] = cc0[t]
                    cc1[d] = cc1[t]
                    cc2[d] = cc2[t]
            return jnp.full((16,), cur - tail, jnp.int32), bq

        issue_slab_a(0)
        issue_slab_b(1)

        def triad_body(ss, carry):
            curv, bq = carry
            issue_slab_c(3 * ss + 2)
            wait_slab_a()
            curv, bq = compact_and_batch(sa0, sa1, sa2, curv, bq)
            issue_slab_a(3 * ss + 3)
            wait_slab_b()
            curv, bq = compact_and_batch(sb0, sb1, sb2, curv, bq)

            @pl.when(ss < _NTRIAD - 1)
            def _():
                issue_slab_b(3 * ss + 4)
            wait_slab_c()
            curv, bq = compact_and_batch(sc0_, sc1_, sc2_, curv, bq)
            return curv, bq

        curv, bq = lax.fori_loop(0, _NTRIAD, triad_body,
                                 (jnp.zeros((16,), jnp.int32), 0))
        wait_slab_a()
        curv, bq = compact_and_batch(sa0, sa1, sa2, curv, bq)
        cur = curv[0]

        @pl.when(cur > 0)
        def _():
            for k in range(_BATCH // 16):
                d = pl.ds(cur + k * 16, 16)
                cc0[d] = jnp.full((16,), _TRASH, jnp.int32)
                cc1[d] = jnp.zeros((16,), jnp.int32)
                cc2[d] = jnp.zeros((16,), jnp.int32)
        bq = lax.fori_loop(0, (cur > 0).astype(jnp.int32),
                           lambda j, b: batch_step(0, b), bq)

        @pl.when(bq >= 2)
        def _():
            @pl.when((bq & 1) == 0)
            def _():
                complete_a(bq)

            @pl.when((bq & 1) == 1)
            def _():
                complete_b(bq)

        @pl.when(bq >= 1)
        def _():
            @pl.when(((bq - 1) & 1) == 0)
            def _():
                complete_a(bq + 1)

            @pl.when(((bq - 1) & 1) == 1)
            def _():
                complete_b(bq + 1)

        plsc.subcore_barrier()
        @pl.when(s < 15)
        def _():
            pltpu.sync_copy(spmem.at[pl.ds(s * 512, 512)],
                            agg.at[pl.ds(lo + s * 512, 512)])

        @pl.when(s == 15)
        def _():
            pltpu.sync_copy(spmem.at[pl.ds(7680, 320)],
                            agg.at[pl.ds(lo + 7680, 320)])
        return 0

    lax.fori_loop(0, _NGRP // 2, pass_body, 0)


def _sc_gather_mul_segsum(h1, h2, i0, i1, i2):
    P = h1.shape[0]
    mesh = plsc.VectorSubcoreMesh(core_axis_name="c", subcore_axis_name="s")
    f = pl.kernel(
        _sc_body,
        out_type=jax.ShapeDtypeStruct((P, HDIM), jnp.float32),
        mesh=mesh,
        compiler_params=pltpu.CompilerParams(needs_layout_passes=False),
        scratch_types=[
            pltpu.VMEM_SHARED((_PG_PAD, HDIM), jnp.float32),
            pltpu.VMEM((_SLAB,), jnp.int32),
            pltpu.VMEM((_SLAB,), jnp.int32),
            pltpu.VMEM((_SLAB,), jnp.int32),
            pltpu.VMEM((_SLAB,), jnp.int32),
            pltpu.VMEM((_SLAB,), jnp.int32),
            pltpu.VMEM((_SLAB,), jnp.int32),
            pltpu.VMEM((_SLAB,), jnp.int32),
            pltpu.VMEM((_SLAB,), jnp.int32),
            pltpu.VMEM((_SLAB,), jnp.int32),
            pltpu.VMEM((_CAP,), jnp.int32),
            pltpu.VMEM((_CAP,), jnp.int32),
            pltpu.VMEM((_CAP,), jnp.int32),
            pltpu.VMEM((_BATCH,), jnp.int32),
            pltpu.VMEM((_BATCH,), jnp.int32),
            pltpu.VMEM((_BATCH,), jnp.int32),
            pltpu.VMEM((_BATCH,), jnp.int32),
            pltpu.VMEM((_BATCH,), jnp.int32),
            pltpu.VMEM((_BATCH,), jnp.int32),
            pltpu.VMEM((_BATCH, HDIM), jnp.float32),
            pltpu.VMEM((_BATCH, HDIM), jnp.float32),
            pltpu.VMEM((_BATCH, HDIM), jnp.float32),
            pltpu.VMEM((_BATCH, HDIM), jnp.float32),
            pltpu.SemaphoreType.DMA,
            pltpu.SemaphoreType.DMA,
            pltpu.SemaphoreType.DMA,
            pltpu.SemaphoreType.DMA,
            pltpu.SemaphoreType.DMA,
            pltpu.SemaphoreType.DMA,
            pltpu.SemaphoreType.DMA,
            pltpu.SemaphoreType.DMA,
            pltpu.SemaphoreType.DMA,
            pltpu.SemaphoreType.DMA,
            pltpu.SemaphoreType.DMA,
            pltpu.SemaphoreType.DMA,
            pltpu.SemaphoreType.DMA,
        ],
    )
    return f(h1, h2, i0, i1, i2)


def kernel(pair_h, tuple_index, W1a, b1a, g1a, be1a, W1b, b1b, W2a, b2a, g2a,
           be2a, W2b, b2b, Wu1, bu1, gu, beu, Wu2, bu2):
    idx0 = tuple_index[0]
    idx1 = tuple_index[1]
    idx2 = tuple_index[2]
    L = W1a.shape[0]
    x2 = pair_h
    for l in range(L):
        h1, h2 = _pair_mlps(x2, W1a[l], b1a[l], g1a[l], be1a[l], W1b[l], b1b[l],
                            W2a[l], b2a[l], g2a[l], be2a[l], W2b[l], b2b[l])
        agg = _sc_gather_mul_segsum(h1, h2, idx0, idx1, idx2)
        x2 = _update(x2, agg, Wu1[l], bu1[l], gu[l], beu[l], Wu2[l], bu2[l])
    return x2


# speedup vs baseline: 1.0910x; 1.0910x over previous
import functools

import jax
import jax.numpy as jnp
from jax import lax
from jax.experimental import pallas as pl
from jax.experimental.pallas import tpu as pltpu
from jax.experimental.pallas import tpu_sc as plsc

HDIM = 128
EPS_BN = 1e-5
EPS_SQRT = 1e-6
BN_SCALE = 1.0 / (1.0 + EPS_BN) ** 0.5

_PREC = jax.lax.Precision.DEFAULT


def _pair_mlps_body(x_ref, w1a_ref, b1a_ref, g1a_ref, be1a_ref, w1b_ref, b1b_ref,
                    w2a_ref, b2a_ref, g2a_ref, be2a_ref, w2b_ref, b2b_ref,
                    h1_ref, h2_ref):
    x = x_ref[...]
    t1 = jnp.dot(x, w1a_ref[...], precision=_PREC) + b1a_ref[...]
    t1 = t1 * (BN_SCALE * g1a_ref[...]) + be1a_ref[...]
    t1 = jnp.maximum(t1, 0.0)
    h1_ref[...] = jnp.dot(t1, w1b_ref[...], precision=_PREC) + b1b_ref[...]
    t2 = jnp.dot(x, w2a_ref[...], precision=_PREC) + b2a_ref[...]
    t2 = t2 * (BN_SCALE * g2a_ref[...]) + be2a_ref[...]
    t2 = jnp.maximum(t2, 0.0)
    h2_ref[...] = jnp.dot(t2, w2b_ref[...], precision=_PREC) + b2b_ref[...]


def _update_body(x_ref, agg_ref, wu1x_ref, wu1a_ref, bu1_ref, gu_ref, beu_ref,
                 wu2_ref, bu2_ref, out_ref):
    x = x_ref[...]
    a = agg_ref[...]
    a = jnp.sqrt(jnp.maximum(a, 0.0) + EPS_SQRT) - jnp.sqrt(jnp.maximum(-a, 0.0) + EPS_SQRT)
    t = (jnp.dot(x, wu1x_ref[...], precision=_PREC)
         + jnp.dot(a, wu1a_ref[...], precision=_PREC) + bu1_ref[...])
    t = t * (BN_SCALE * gu_ref[...]) + beu_ref[...]
    t = jnp.maximum(t, 0.0)
    out_ref[...] = jnp.dot(t, wu2_ref[...], precision=_PREC) + bu2_ref[...] + x


def _full(shape):
    return pl.BlockSpec(shape, lambda i: (0,) * len(shape))


def _pair_mlps(x, w1a, b1a, g1a, be1a, w1b, b1b, w2a, b2a, g2a, be2a, w2b, b2b):
    P = x.shape[0]
    BP = 2000
    grid = (P // BP,)
    row = pl.BlockSpec((BP, HDIM), lambda i: (i, 0))
    mat = _full((HDIM, HDIM))
    vec = _full((HDIM,))
    return pl.pallas_call(
        _pair_mlps_body,
        grid=grid,
        in_specs=[row, mat, vec, vec, vec, mat, vec, mat, vec, vec, vec, mat, vec],
        out_specs=[row, row],
        out_shape=[jax.ShapeDtypeStruct((P, HDIM), jnp.float32)] * 2,
    )(x, w1a, b1a, g1a, be1a, w1b, b1b, w2a, b2a, g2a, be2a, w2b, b2b)


def _update(x, agg, wu1, bu1, gu, beu, wu2, bu2):
    P = x.shape[0]
    BP = 2000
    grid = (P // BP,)
    row = pl.BlockSpec((BP, HDIM), lambda i: (i, 0))
    mat = _full((HDIM, HDIM))
    vec = _full((HDIM,))
    return pl.pallas_call(
        _update_body,
        grid=grid,
        in_specs=[row, row, mat, mat, vec, vec, vec, mat, vec],
        out_specs=row,
        out_shape=jax.ShapeDtypeStruct((P, HDIM), jnp.float32),
    )(x, agg, wu1[:HDIM], wu1[HDIM:], bu1, gu, beu, wu2, bu2)


_T_TOTAL = 1280000
_T_TILE = _T_TOTAL // 16
_SLAB = 2000
_NVREG = _SLAB // 16
_UNROLL = 5
_NTRIAD = 13
_NGRP = 20
_PG = 160000 // _NGRP
_PG_PAD = 8192
_TRASH = _PG
_CAP = 2176
_BATCH = 64
_BSH = 6


def _sc_body(h1, h2, i0, i1, i2, agg, spmem,
             sa0, sa1, sa2, sb0, sb1, sb2, sc0_, sc1_, sc2_,
             cc0, cc1, cc2,
             b0A, b1A, b2A, b0B, b1B, b2B,
             g1A, g2A, g1B, g2B,
             ma0, ma1, ma2, mb0, mb1, mb2, mc0, mc1, mc2,
             mg1A, mg2A, mg1B, mg2B):
    c = lax.axis_index("c")
    s = lax.axis_index("s")
    tile_base = s * _T_TILE

    def issue_slab_a(sl):
        base = tile_base + sl * _SLAB
        pltpu.async_copy(i0.at[pl.ds(base, _SLAB)], sa0, ma0)
        pltpu.async_copy(i1.at[pl.ds(base, _SLAB)], sa1, ma1)
        pltpu.async_copy(i2.at[pl.ds(base, _SLAB)], sa2, ma2)

    def wait_slab_a():
        pltpu.make_async_copy(i0.at[pl.ds(0, _SLAB)], sa0, ma0).wait()
        pltpu.make_async_copy(i1.at[pl.ds(0, _SLAB)], sa1, ma1).wait()
        pltpu.make_async_copy(i2.at[pl.ds(0, _SLAB)], sa2, ma2).wait()

    def issue_slab_b(sl):
        base = tile_base + sl * _SLAB
        pltpu.async_copy(i0.at[pl.ds(base, _SLAB)], sb0, mb0)
        pltpu.async_copy(i1.at[pl.ds(base, _SLAB)], sb1, mb1)
        pltpu.async_copy(i2.at[pl.ds(base, _SLAB)], sb2, mb2)

    def wait_slab_b():
        pltpu.make_async_copy(i0.at[pl.ds(0, _SLAB)], sb0, mb0).wait()
        pltpu.make_async_copy(i1.at[pl.ds(0, _SLAB)], sb1, mb1).wait()
        pltpu.make_async_copy(i2.at[pl.ds(0, _SLAB)], sb2, mb2).wait()

    def issue_slab_c(sl):
        base = tile_base + sl * _SLAB
        pltpu.async_copy(i0.at[pl.ds(base, _SLAB)], sc0_, mc0)
        pltpu.async_copy(i1.at[pl.ds(base, _SLAB)], sc1_, mc1)
        pltpu.async_copy(i2.at[pl.ds(base, _SLAB)], sc2_, mc2)

    def wait_slab_c():
        pltpu.make_async_copy(i0.at[pl.ds(0, _SLAB)], sc0_, mc0).wait()
        pltpu.make_async_copy(i1.at[pl.ds(0, _SLAB)], sc1_, mc1).wait()
        pltpu.make_async_copy(i2.at[pl.ds(0, _SLAB)], sc2_, mc2).wait()

    def complete_a(bq):
        pltpu.make_async_copy(h1.at[b1A], g1A, mg1A).wait()
        pltpu.make_async_copy(h2.at[b2A], g2A, mg2A).wait()

        def mul_row(r, _):
            for u in range(4):
                for q in range(8):
                    d = pl.ds(q * 16, 16)
                    g1A[r * 4 + u, d] = g1A[r * 4 + u, d] * g2A[r * 4 + u, d]
            return 0
        lax.fori_loop(0, _BATCH // 4, mul_row, 0)
        pltpu.sync_copy(g1A, spmem.at[b0A], add=True)

    def complete_b(bq):
        pltpu.make_async_copy(h1.at[b1B], g1B, mg1B).wait()
        pltpu.make_async_copy(h2.at[b2B], g2B, mg2B).wait()

        def mul_row(r, _):
            for u in range(4):
                for q in range(8):
                    d = pl.ds(q * 16, 16)
                    g1B[r * 4 + u, d] = g1B[r * 4 + u, d] * g2B[r * 4 + u, d]
            return 0
        lax.fori_loop(0, _BATCH // 4, mul_row, 0)
        pltpu.sync_copy(g1B, spmem.at[b0B], add=True)

    def issue_a(start):
        for k in range(_BATCH // 16):
            d = pl.ds(k * 16, 16)
            t = pl.ds(start + k * 16, 16)
            b0A[d] = cc0[t]
            b1A[d] = cc1[t]
            b2A[d] = cc2[t]
        pltpu.async_copy(h1.at[b1A], g1A, mg1A)
        pltpu.async_copy(h2.at[b2A], g2A, mg2A)

    def issue_b(start):
        for k in range(_BATCH // 16):
            d = pl.ds(k * 16, 16)
            t = pl.ds(start + k * 16, 16)
            b0B[d] = cc0[t]
            b1B[d] = cc1[t]
            b2B[d] = cc2[t]
        pltpu.async_copy(h1.at[b1B], g1B, mg1B)
        pltpu.async_copy(h2.at[b2B], g2B, mg2B)

    def batch_step(start, bq):
        @pl.when((bq & 1) == 0)
        def _():
            @pl.when(bq >= 2)
            def _():
                complete_a(bq)
            issue_a(start)

        @pl.when((bq & 1) == 1)
        def _():
            @pl.when(bq >= 2)
            def _():
                complete_b(bq)
            issue_b(start)
        return bq + 1

    def pass_body(g, _):
        lo = (c * (_NGRP // 2) + g) * _PG

        def zero_row(r, _):
            for q in range(8):
                g1A[r, pl.ds(q * 16, 16)] = jnp.zeros((16,), jnp.float32)
            return 0
        lax.fori_loop(0, _BATCH, zero_row, 0)
        for k in range(512 // _BATCH):
            pltpu.sync_copy(g1A, spmem.at[pl.ds(s * 512 + k * _BATCH, _BATCH)])
        plsc.subcore_barrier()

        def compact_and_batch(s0, s1, s2, cur, bq):
            def vreg_body(i, cur):
                for u in range(_UNROLL):
                    off = pl.ds((i * _UNROLL + u) * 16, 16)
                    d0 = s0[off] - lo
                    m = d0.astype(jnp.uint32) < jnp.uint32(_PG)
                    plsc.store_compressed(cc0.at[pl.ds(cur, 16)], d0, mask=m)
                    plsc.store_compressed(cc1.at[pl.ds(cur, 16)], s1[off],
                                          mask=m)
                    plsc.store_compressed(cc2.at[pl.ds(cur, 16)], s2[off],
                                          mask=m)
                    cur = cur + plsc.all_reduce_population_count(m)[0]
                return cur
            cur = lax.fori_loop(0, _NVREG // _UNROLL, vreg_body, cur)

            nfull = cur >> _BSH

            def bb(j, bq):
                return batch_step(j * _BATCH, bq)
            bq = lax.fori_loop(0, nfull, bb, bq)

            tail = nfull << _BSH

            @pl.when(nfull > 0)
            def _():
                for k in range(_BATCH // 16):
                    d = pl.ds(k * 16, 16)
                    t = pl.ds(tail + k * 16, 16)
                    cc0[d] = cc0[t]
                    cc1[d] = cc1[t]
                    cc2[d] = cc2[t]
            return cur - tail, bq

        issue_slab_a(0)
        issue_slab_b(1)

        def triad_body(ss, carry):
            cur, bq = carry
            issue_slab_c(3 * ss + 2)
            wait_slab_a()
            cur, bq = compact_and_batch(sa0, sa1, sa2, cur, bq)
            issue_slab_a(3 * ss + 3)
            wait_slab_b()
            cur, bq = compact_and_batch(sb0, sb1, sb2, cur, bq)

            @pl.when(ss < _NTRIAD - 1)
            def _():
                issue_slab_b(3 * ss + 4)
            wait_slab_c()
            cur, bq = compact_and_batch(sc0_, sc1_, sc2_, cur, bq)
            return cur, bq

        cur, bq = lax.fori_loop(0, _NTRIAD, triad_body, (0, 0))
        wait_slab_a()
        cur, bq = compact_and_batch(sa0, sa1, sa2, cur, bq)

        @pl.when(cur > 0)
        def _():
            for k in range(_BATCH // 16):
                d = pl.ds(cur + k * 16, 16)
                cc0[d] = jnp.full((16,), _TRASH, jnp.int32)
                cc1[d] = jnp.zeros((16,), jnp.int32)
                cc2[d] = jnp.zeros((16,), jnp.int32)
        bq = lax.fori_loop(0, (cur > 0).astype(jnp.int32),
                           lambda j, b: batch_step(0, b), bq)

        @pl.when(bq >= 2)
        def _():
            @pl.when((bq & 1) == 0)
            def _():
                complete_a(bq)

            @pl.when((bq & 1) == 1)
            def _():
                complete_b(bq)

        @pl.when(bq >= 1)
        def _():
            @pl.when(((bq - 1) & 1) == 0)
            def _():
                complete_a(bq + 1)

            @pl.when(((bq - 1) & 1) == 1)
            def _():
                complete_b(bq + 1)

        plsc.subcore_barrier()
        @pl.when(s < 15)
        def _():
            pltpu.sync_copy(spmem.at[pl.ds(s * 512, 512)],
                            agg.at[pl.ds(lo + s * 512, 512)])

        @pl.when(s == 15)
        def _():
            pltpu.sync_copy(spmem.at[pl.ds(7680, 320)],
                            agg.at[pl.ds(lo + 7680, 320)])
        return 0

    lax.fori_loop(0, _NGRP // 2, pass_body, 0)


def _sc_gather_mul_segsum(h1, h2, i0, i1, i2):
    P = h1.shape[0]
    mesh = plsc.VectorSubcoreMesh(core_axis_name="c", subcore_axis_name="s")
    f = pl.kernel(
        _sc_body,
        out_type=jax.ShapeDtypeStruct((P, HDIM), jnp.float32),
        mesh=mesh,
        compiler_params=pltpu.CompilerParams(needs_layout_passes=False),
        scratch_types=[
            pltpu.VMEM_SHARED((_PG_PAD, HDIM), jnp.float32),
            pltpu.VMEM((_SLAB,), jnp.int32),
            pltpu.VMEM((_SLAB,), jnp.int32),
            pltpu.VMEM((_SLAB,), jnp.int32),
            pltpu.VMEM((_SLAB,), jnp.int32),
            pltpu.VMEM((_SLAB,), jnp.int32),
            pltpu.VMEM((_SLAB,), jnp.int32),
            pltpu.VMEM((_SLAB,), jnp.int32),
            pltpu.VMEM((_SLAB,), jnp.int32),
            pltpu.VMEM((_SLAB,), jnp.int32),
            pltpu.VMEM((_CAP,), jnp.int32),
            pltpu.VMEM((_CAP,), jnp.int32),
            pltpu.VMEM((_CAP,), jnp.int32),
            pltpu.VMEM((_BATCH,), jnp.int32),
            pltpu.VMEM((_BATCH,), jnp.int32),
            pltpu.VMEM((_BATCH,), jnp.int32),
            pltpu.VMEM((_BATCH,), jnp.int32),
            pltpu.VMEM((_BATCH,), jnp.int32),
            pltpu.VMEM((_BATCH,), jnp.int32),
            pltpu.VMEM((_BATCH, HDIM), jnp.float32),
            pltpu.VMEM((_BATCH, HDIM), jnp.float32),
            pltpu.VMEM((_BATCH, HDIM), jnp.float32),
            pltpu.VMEM((_BATCH, HDIM), jnp.float32),
            pltpu.SemaphoreType.DMA,
            pltpu.SemaphoreType.DMA,
            pltpu.SemaphoreType.DMA,
            pltpu.SemaphoreType.DMA,
            pltpu.SemaphoreType.DMA,
            pltpu.SemaphoreType.DMA,
            pltpu.SemaphoreType.DMA,
            pltpu.SemaphoreType.DMA,
            pltpu.SemaphoreType.DMA,
            pltpu.SemaphoreType.DMA,
            pltpu.SemaphoreType.DMA,
            pltpu.SemaphoreType.DMA,
            pltpu.SemaphoreType.DMA,
        ],
    )
    return f(h1, h2, i0, i1, i2)


def kernel(pair_h, tuple_index, W1a, b1a, g1a, be1a, W1b, b1b, W2a, b2a, g2a,
           be2a, W2b, b2b, Wu1, bu1, gu, beu, Wu2, bu2):
    idx0 = tuple_index[0]
    idx1 = tuple_index[1]
    idx2 = tuple_index[2]
    L = W1a.shape[0]
    x2 = pair_h
    for l in range(L):
        h1, h2 = _pair_mlps(x2, W1a[l], b1a[l], g1a[l], be1a[l], W1b[l], b1b[l],
                            W2a[l], b2a[l], g2a[l], be2a[l], W2b[l], b2b[l])
        agg = _sc_gather_mul_segsum(h1, h2, idx0, idx1, idx2)
        x2 = _update(x2, agg, Wu1[l], bu1[l], gu[l], beu[l], Wu2[l], bu2[l])
    return x2
